# baseline (speedup 1.0000x reference)
import functools

import jax
import jax.numpy as jnp
from jax import lax
from jax.experimental import pallas as pl
from jax.experimental.pallas import tpu as pltpu
from jax.experimental.pallas import tpu_sc as plsc

T = 2048
D = 1024
F = 2048
E = 8
KTOP = 2

BM = 128
NB = T * KTOP // BM + E
S = NB * BM


def _router_body(x_ref, wr_ref, pos0_ref, pos1_ref, g0_ref, g1_ref, gb_ref):
    x = x_ref[...]
    logits = jnp.dot(x, wr_ref[...], preferred_element_type=jnp.float32)
    iota_e = lax.broadcasted_iota(jnp.int32, (T, E), 1)
    m0 = jnp.max(logits, axis=-1, keepdims=True)
    idx0 = jnp.min(jnp.where(logits == m0, iota_e, E), axis=-1, keepdims=True)
    masked = jnp.where(iota_e == idx0, -jnp.inf, logits)
    m1 = jnp.max(masked, axis=-1, keepdims=True)
    idx1 = jnp.min(jnp.where(masked == m1, iota_e, E), axis=-1, keepdims=True)
    d = jnp.exp(m1 - m0)
    g0_ref[...] = jnp.broadcast_to(1.0 / (1.0 + d), (T, 16))
    g1_ref[...] = jnp.broadcast_to(d / (1.0 + d), (T, 16))
    oh0 = (iota_e == idx0).astype(jnp.float32)
    oh1 = (iota_e == idx1).astype(jnp.float32)
    onehot = jnp.concatenate([oh0, oh1], axis=0)
    CH = 512
    r = lax.broadcasted_iota(jnp.int32, (CH, CH), 0)
    c = lax.broadcasted_iota(jnp.int32, (CH, CH), 1)
    tri = (r >= c).astype(jnp.float32)
    carry = jnp.zeros((1, E), dtype=jnp.float32)
    chunks = []
    for i in range(2 * T // CH):
        cs = jnp.dot(tri, onehot[i * CH:(i + 1) * CH], preferred_element_type=jnp.float32) + carry
        chunks.append(cs)
        carry = cs[CH - 1:CH, :]
    ccum = jnp.concatenate(chunks, axis=0)
    counts = ccum[2 * T - 1:2 * T, :]
    nb = jnp.floor((counts + (BM - 1)) * (1.0 / BM))
    r8 = lax.broadcasted_iota(jnp.int32, (E, E), 0)
    c8 = lax.broadcasted_iota(jnp.int32, (E, E), 1)
    m_lt = (r8 < c8).astype(jnp.float32)
    m_le = (r8 <= c8).astype(jnp.float32)
    bases = BM * jnp.dot(nb, m_lt, preferred_element_type=jnp.float32)
    incl_nb = jnp.dot(nb, m_le, preferred_element_type=jnp.float32)
    pos = jnp.sum(onehot * (ccum - 1.0 + bases), axis=-1, keepdims=True)
    pos = pos.astype(jnp.int32)
    pos0_ref[...] = pos[:T]
    pos1_ref[...] = pos[T:]
    jmat = lax.broadcasted_iota(jnp.int32, (NB, E), 0).astype(jnp.float32)
    gb = jnp.sum((jnp.broadcast_to(incl_nb, (NB, E)) <= jmat).astype(jnp.float32),
                 axis=-1, keepdims=True)
    gb_ref[...] = jnp.minimum(gb, E - 1).astype(jnp.int32)


def _router_dispatch(x, wr, interpret=False):
    return pl.pallas_call(
        _router_body,
        out_shape=[
            jax.ShapeDtypeStruct((T, 1), jnp.int32),
            jax.ShapeDtypeStruct((T, 1), jnp.int32),
            jax.ShapeDtypeStruct((T, 16), jnp.float32),
            jax.ShapeDtypeStruct((T, 16), jnp.float32),
            jax.ShapeDtypeStruct((NB, 1), jnp.int32),
        ],
        interpret=interpret,
    )(x, wr)


def _gmm_body(gb_ref, xs_ref, wg_ref, wu_ref, wd_ref, o_ref,
              wgc, wuc, wdc, last):
    j = pl.program_id(0)
    e = gb_ref[j]

    @pl.when((j == 0) | (e != last[0]))
    def _():
        wgc[...] = wg_ref[0].astype(jnp.bfloat16)
        wuc[...] = wu_ref[0].astype(jnp.bfloat16)
        wdc[...] = wd_ref[0].astype(jnp.bfloat16)
        last[0] = e

    xb = xs_ref[...].astype(jnp.bfloat16)
    g = jnp.dot(xb, wgc[...], preferred_element_type=jnp.float32)
    u = jnp.dot(xb, wuc[...], preferred_element_type=jnp.float32)
    h = (g * jax.nn.sigmoid(g) * u).astype(jnp.bfloat16)
    o_ref[...] = jnp.dot(h, wdc[...], preferred_element_type=jnp.float32)


def _gmm(gb, xs, wg, wu, wd, interpret=False):
    grid_spec = pltpu.PrefetchScalarGridSpec(
        num_scalar_prefetch=1,
        grid=(NB,),
        in_specs=[
            pl.BlockSpec((BM, D), lambda j, gb: (j, 0)),
            pl.BlockSpec((1, D, F), lambda j, gb: (gb[j], 0, 0)),
            pl.BlockSpec((1, D, F), lambda j, gb: (gb[j], 0, 0)),
            pl.BlockSpec((1, F, D), lambda j, gb: (gb[j], 0, 0)),
        ],
        out_specs=pl.BlockSpec((BM, D), lambda j, gb: (j, 0)),
        scratch_shapes=[
            pltpu.VMEM((D, F), jnp.bfloat16),
            pltpu.VMEM((D, F), jnp.bfloat16),
            pltpu.VMEM((F, D), jnp.bfloat16),
            pltpu.SMEM((1,), jnp.int32),
        ],
    )
    return pl.pallas_call(
        _gmm_body,
        grid_spec=grid_spec,
        out_shape=jax.ShapeDtypeStruct((S, D), jnp.float32),
        compiler_params=pltpu.CompilerParams(
            dimension_semantics=("arbitrary",),
            vmem_limit_bytes=112 * 1024 * 1024),
        interpret=interpret,
    )(gb, xs, wg, wu, wd)


NW = 32
TPW = T // NW


def _nc():
    return plsc.get_sparse_core_info().num_cores


def _dispatch_scatter(x, pos0, pos1):
    nc = _nc()
    mesh = plsc.VectorSubcoreMesh(core_axis_name="c", subcore_axis_name="s")

    @functools.partial(
        pl.kernel,
        out_type=jax.ShapeDtypeStruct((S, D), jnp.float32),
        mesh=mesh,
        scratch_types=[
            pltpu.VMEM((TPW, D), jnp.float32),
            pltpu.VMEM((TPW,), jnp.int32),
            pltpu.VMEM((TPW,), jnp.int32),
            pltpu.SemaphoreType.DMA,
        ],
    )
    def k(x_hbm, pos0_hbm, pos1_hbm, xs_hbm, xbuf, i0, i1, sem):
        w = lax.axis_index("s") * nc + lax.axis_index("c")
        base = w * TPW
        pltpu.sync_copy(x_hbm.at[pl.ds(base, TPW)], xbuf)
        pltpu.sync_copy(pos0_hbm.at[pl.ds(base, TPW)], i0)
        pltpu.sync_copy(pos1_hbm.at[pl.ds(base, TPW)], i1)
        pltpu.async_copy(xbuf, xs_hbm.at[i0], sem).wait()
        pltpu.async_copy(xbuf, xs_hbm.at[i1], sem).wait()

    return k(x, pos0, pos1)


_CH = 32


def _combine(o, pos0, pos1, g0, g1):
    nc = _nc()
    mesh = plsc.VectorSubcoreMesh(core_axis_name="c", subcore_axis_name="s")

    @functools.partial(
        pl.kernel,
        out_type=jax.ShapeDtypeStruct((T, D), jnp.float32),
        mesh=mesh,
        scratch_types=[
            pltpu.VMEM((_CH, D), jnp.float32),
            pltpu.VMEM((_CH, D), jnp.float32),
            pltpu.VMEM((_CH,), jnp.int32),
            pltpu.VMEM((_CH,), jnp.int32),
            pltpu.VMEM((_CH, 16), jnp.float32),
            pltpu.VMEM((_CH, 16), jnp.float32),
            pltpu.SemaphoreType.DMA,
        ],
    )
    def k(o_hbm, pos0_hbm, pos1_hbm, g0_hbm, g1_hbm, out_hbm,
          b0, b1, i0, i1, gg0, gg1, sem):
        w = lax.axis_index("s") * nc + lax.axis_index("c")
        for cidx in range(TPW // _CH):
            base = w * TPW + cidx * _CH
            pltpu.sync_copy(pos0_hbm.at[pl.ds(base, _CH)], i0)
            pltpu.sync_copy(pos1_hbm.at[pl.ds(base, _CH)], i1)
            pltpu.sync_copy(g0_hbm.at[pl.ds(base, _CH)], gg0)
            pltpu.sync_copy(g1_hbm.at[pl.ds(base, _CH)], gg1)
            pltpu.async_copy(o_hbm.at[i0], b0, sem).wait()
            pltpu.async_copy(o_hbm.at[i1], b1, sem).wait()
            for tt in range(_CH):
                gv0 = gg0[tt, :]
                gv1 = gg1[tt, :]

                def body(jj, _, tt=tt, gv0=gv0, gv1=gv1):
                    sl = pl.ds(jj * 16, 16)
                    b0[tt, sl] = gv0 * b0[tt, sl] + gv1 * b1[tt, sl]
                    return 0

                lax.fori_loop(0, D // 16, body, 0)
            pltpu.sync_copy(b0, out_hbm.at[pl.ds(base, _CH)])

    return k(o, pos0, pos1, g0, g1)


def kernel(x_TD, w_router_DE, w_gate_EDF, w_up_EDF, w_down_EFD):
    pos0, pos1, g0, g1, gb = _router_dispatch(x_TD, w_router_DE)
    pos0 = pos0.reshape(T)
    pos1 = pos1.reshape(T)
    gb = gb.reshape(NB)
    xs = _dispatch_scatter(x_TD, pos0, pos1)
    o = _gmm(gb, xs, w_gate_EDF, w_up_EDF, w_down_EFD)
    return _combine(o, pos0, pos1, g0, g1)


# speedup vs baseline: 1.0467x; 1.0467x over previous
import functools

import jax
import jax.numpy as jnp
from jax import lax
from jax.experimental import pallas as pl
from jax.experimental.pallas import tpu as pltpu
from jax.experimental.pallas import tpu_sc as plsc

T = 2048
D = 1024
F = 2048
E = 8
KTOP = 2

BM = 256
NB = T * KTOP // BM + E
S = NB * BM


def _router_body(x_ref, wr_ref, pos0_ref, pos1_ref, g0_ref, g1_ref, gb_ref):
    x = x_ref[...]
    logits = jnp.dot(x, wr_ref[...], preferred_element_type=jnp.float32)
    iota_e = lax.broadcasted_iota(jnp.int32, (T, E), 1)
    m0 = jnp.max(logits, axis=-1, keepdims=True)
    idx0 = jnp.min(jnp.where(logits == m0, iota_e, E), axis=-1, keepdims=True)
    masked = jnp.where(iota_e == idx0, -jnp.inf, logits)
    m1 = jnp.max(masked, axis=-1, keepdims=True)
    idx1 = jnp.min(jnp.where(masked == m1, iota_e, E), axis=-1, keepdims=True)
    d = jnp.exp(m1 - m0)
    g0_ref[...] = jnp.broadcast_to(1.0 / (1.0 + d), (T, 16))
    g1_ref[...] = jnp.broadcast_to(d / (1.0 + d), (T, 16))
    oh0 = (iota_e == idx0).astype(jnp.float32)
    oh1 = (iota_e == idx1).astype(jnp.float32)
    onehot = jnp.concatenate([oh0, oh1], axis=0)
    CH = 512
    r = lax.broadcasted_iota(jnp.int32, (CH, CH), 0)
    c = lax.broadcasted_iota(jnp.int32, (CH, CH), 1)
    tri = (r >= c).astype(jnp.float32)
    carry = jnp.zeros((1, E), dtype=jnp.float32)
    chunks = []
    for i in range(2 * T // CH):
        cs = jnp.dot(tri, onehot[i * CH:(i + 1) * CH], preferred_element_type=jnp.float32) + carry
        chunks.append(cs)
        carry = cs[CH - 1:CH, :]
    ccum = jnp.concatenate(chunks, axis=0)
    counts = ccum[2 * T - 1:2 * T, :]
    nb = jnp.floor((counts + (BM - 1)) * (1.0 / BM))
    r8 = lax.broadcasted_iota(jnp.int32, (E, E), 0)
    c8 = lax.broadcasted_iota(jnp.int32, (E, E), 1)
    m_lt = (r8 < c8).astype(jnp.float32)
    m_le = (r8 <= c8).astype(jnp.float32)
    bases = BM * jnp.dot(nb, m_lt, preferred_element_type=jnp.float32)
    incl_nb = jnp.dot(nb, m_le, preferred_element_type=jnp.float32)
    pos = jnp.sum(onehot * (ccum - 1.0 + bases), axis=-1, keepdims=True)
    pos = pos.astype(jnp.int32)
    pos0_ref[...] = pos[:T]
    pos1_ref[...] = pos[T:]
    jmat = lax.broadcasted_iota(jnp.int32, (NB, E), 0).astype(jnp.float32)
    gb = jnp.sum((jnp.broadcast_to(incl_nb, (NB, E)) <= jmat).astype(jnp.float32),
                 axis=-1, keepdims=True)
    gb_ref[...] = jnp.minimum(gb, E - 1).astype(jnp.int32)


def _router_dispatch(x, wr, interpret=False):
    return pl.pallas_call(
        _router_body,
        out_shape=[
            jax.ShapeDtypeStruct((T, 1), jnp.int32),
            jax.ShapeDtypeStruct((T, 1), jnp.int32),
            jax.ShapeDtypeStruct((T, 16), jnp.float32),
            jax.ShapeDtypeStruct((T, 16), jnp.float32),
            jax.ShapeDtypeStruct((NB, 1), jnp.int32),
        ],
        interpret=interpret,
    )(x, wr)


def _gmm_body(gb_ref, xs_ref, wg_ref, wu_ref, wd_ref, o_ref):
    xb = xs_ref[...]
    g = jnp.dot(xb, wg_ref[0], preferred_element_type=jnp.float32)
    u = jnp.dot(xb, wu_ref[0], preferred_element_type=jnp.float32)
    h = g * jax.nn.sigmoid(g) * u
    o_ref[...] = jnp.dot(h, wd_ref[0], preferred_element_type=jnp.float32)


def _gmm(gb, xs, wg, wu, wd, interpret=False):
    grid_spec = pltpu.PrefetchScalarGridSpec(
        num_scalar_prefetch=1,
        grid=(NB,),
        in_specs=[
            pl.BlockSpec((BM, D), lambda j, gb: (j, 0)),
            pl.BlockSpec((1, D, F), lambda j, gb: (gb[j], 0, 0)),
            pl.BlockSpec((1, D, F), lambda j, gb: (gb[j], 0, 0)),
            pl.BlockSpec((1, F, D), lambda j, gb: (gb[j], 0, 0)),
        ],
        out_specs=pl.BlockSpec((BM, D), lambda j, gb: (j, 0)),
    )
    return pl.pallas_call(
        _gmm_body,
        grid_spec=grid_spec,
        out_shape=jax.ShapeDtypeStruct((S, D), jnp.float32),
        compiler_params=pltpu.CompilerParams(
            dimension_semantics=("arbitrary",),
            vmem_limit_bytes=112 * 1024 * 1024),
        interpret=interpret,
    )(gb, xs, wg, wu, wd)


NW = 32
TPW = T // NW


def _nc():
    return plsc.get_sparse_core_info().num_cores


def _dispatch_scatter(x, pos0, pos1):
    nc = _nc()
    mesh = plsc.VectorSubcoreMesh(core_axis_name="c", subcore_axis_name="s")

    @functools.partial(
        pl.kernel,
        out_type=jax.ShapeDtypeStruct((S, D), jnp.float32),
        mesh=mesh,
        scratch_types=[
            pltpu.VMEM((TPW, D), jnp.float32),
            pltpu.VMEM((TPW,), jnp.int32),
            pltpu.VMEM((TPW,), jnp.int32),
            pltpu.SemaphoreType.DMA,
        ],
    )
    def k(x_hbm, pos0_hbm, pos1_hbm, xs_hbm, xbuf, i0, i1, sem):
        w = lax.axis_index("s") * nc + lax.axis_index("c")
        base = w * TPW
        pltpu.sync_copy(x_hbm.at[pl.ds(base, TPW)], xbuf)
        pltpu.sync_copy(pos0_hbm.at[pl.ds(base, TPW)], i0)
        pltpu.sync_copy(pos1_hbm.at[pl.ds(base, TPW)], i1)
        pltpu.async_copy(xbuf, xs_hbm.at[i0], sem).wait()
        pltpu.async_copy(xbuf, xs_hbm.at[i1], sem).wait()

    return k(x, pos0, pos1)


_CH = 32


def _combine(o, pos0, pos1, g0, g1):
    nc = _nc()
    mesh = plsc.VectorSubcoreMesh(core_axis_name="c", subcore_axis_name="s")

    @functools.partial(
        pl.kernel,
        out_type=jax.ShapeDtypeStruct((T, D), jnp.float32),
        mesh=mesh,
        scratch_types=[
            pltpu.VMEM((_CH, D), jnp.float32),
            pltpu.VMEM((_CH, D), jnp.float32),
            pltpu.VMEM((_CH,), jnp.int32),
            pltpu.VMEM((_CH,), jnp.int32),
            pltpu.VMEM((_CH, 16), jnp.float32),
            pltpu.VMEM((_CH, 16), jnp.float32),
            pltpu.SemaphoreType.DMA,
        ],
    )
    def k(o_hbm, pos0_hbm, pos1_hbm, g0_hbm, g1_hbm, out_hbm,
          b0, b1, i0, i1, gg0, gg1, sem):
        w = lax.axis_index("s") * nc + lax.axis_index("c")
        for cidx in range(TPW // _CH):
            base = w * TPW + cidx * _CH
            pltpu.sync_copy(pos0_hbm.at[pl.ds(base, _CH)], i0)
            pltpu.sync_copy(pos1_hbm.at[pl.ds(base, _CH)], i1)
            pltpu.sync_copy(g0_hbm.at[pl.ds(base, _CH)], gg0)
            pltpu.sync_copy(g1_hbm.at[pl.ds(base, _CH)], gg1)
            pltpu.async_copy(o_hbm.at[i0], b0, sem).wait()
            pltpu.async_copy(o_hbm.at[i1], b1, sem).wait()
            for tt in range(_CH):
                gv0 = gg0[tt, :]
                gv1 = gg1[tt, :]

                def body(jj, _, tt=tt, gv0=gv0, gv1=gv1):
                    sl = pl.ds(jj * 16, 16)
                    b0[tt, sl] = gv0 * b0[tt, sl] + gv1 * b1[tt, sl]
                    return 0

                lax.fori_loop(0, D // 16, body, 0)
            pltpu.sync_copy(b0, out_hbm.at[pl.ds(base, _CH)])

    return k(o, pos0, pos1, g0, g1)


def kernel(x_TD, w_router_DE, w_gate_EDF, w_up_EDF, w_down_EFD):
    pos0, pos1, g0, g1, gb = _router_dispatch(x_TD, w_router_DE)
    pos0 = pos0.reshape(T)
    pos1 = pos1.reshape(T)
    gb = gb.reshape(NB)
    xs = _dispatch_scatter(x_TD, pos0, pos1)
    o = _gmm(gb, xs, w_gate_EDF, w_up_EDF, w_down_EFD)
    return _combine(o, pos0, pos1, g0, g1)


# speedup vs baseline: 1.0699x; 1.0221x over previous
import functools

import jax
import jax.numpy as jnp
from jax import lax
from jax.experimental import pallas as pl
from jax.experimental.pallas import tpu as pltpu
from jax.experimental.pallas import tpu_sc as plsc

T = 2048
D = 1024
F = 2048
E = 8
KTOP = 2

BM = 256
NB = T * KTOP // BM + E
S = NB * BM


def _router_body(x_ref, wr_ref, pos0_ref, pos1_ref, g0_ref, g1_ref, gb_ref):
    x = x_ref[...]
    logits = jnp.dot(x, wr_ref[...], preferred_element_type=jnp.float32)
    iota_e = lax.broadcasted_iota(jnp.int32, (T, E), 1)
    m0 = jnp.max(logits, axis=-1, keepdims=True)
    idx0 = jnp.min(jnp.where(logits == m0, iota_e, E), axis=-1, keepdims=True)
    masked = jnp.where(iota_e == idx0, -jnp.inf, logits)
    m1 = jnp.max(masked, axis=-1, keepdims=True)
    idx1 = jnp.min(jnp.where(masked == m1, iota_e, E), axis=-1, keepdims=True)
    d = jnp.exp(m1 - m0)
    g0_ref[...] = jnp.broadcast_to(1.0 / (1.0 + d), (T, 16))
    g1_ref[...] = jnp.broadcast_to(d / (1.0 + d), (T, 16))
    oh0 = (iota_e == idx0).astype(jnp.float32)
    oh1 = (iota_e == idx1).astype(jnp.float32)
    onehot = jnp.concatenate([oh0, oh1], axis=0)
    CH = 512
    r = lax.broadcasted_iota(jnp.int32, (CH, CH), 0)
    c = lax.broadcasted_iota(jnp.int32, (CH, CH), 1)
    tri = (r >= c).astype(jnp.float32)
    carry = jnp.zeros((1, E), dtype=jnp.float32)
    chunks = []
    for i in range(2 * T // CH):
        cs = jnp.dot(tri, onehot[i * CH:(i + 1) * CH], preferred_element_type=jnp.float32) + carry
        chunks.append(cs)
        carry = cs[CH - 1:CH, :]
    ccum = jnp.concatenate(chunks, axis=0)
    counts = ccum[2 * T - 1:2 * T, :]
    nb = jnp.floor((counts + (BM - 1)) * (1.0 / BM))
    r8 = lax.broadcasted_iota(jnp.int32, (E, E), 0)
    c8 = lax.broadcasted_iota(jnp.int32, (E, E), 1)
    m_lt = (r8 < c8).astype(jnp.float32)
    m_le = (r8 <= c8).astype(jnp.float32)
    bases = BM * jnp.dot(nb, m_lt, preferred_element_type=jnp.float32)
    incl_nb = jnp.dot(nb, m_le, preferred_element_type=jnp.float32)
    pos = jnp.sum(onehot * (ccum - 1.0 + bases), axis=-1, keepdims=True)
    pos = pos.astype(jnp.int32)
    pos0_ref[...] = pos[:T]
    pos1_ref[...] = pos[T:]
    jmat = lax.broadcasted_iota(jnp.int32, (NB, E), 0).astype(jnp.float32)
    gb = jnp.sum((jnp.broadcast_to(incl_nb, (NB, E)) <= jmat).astype(jnp.float32),
                 axis=-1, keepdims=True)
    gb_ref[0:1] = incl_nb[:, E - 1:E].astype(jnp.int32)
    gb_ref[1:] = jnp.minimum(gb, E - 1).astype(jnp.int32)


def _router_dispatch(x, wr, interpret=False):
    return pl.pallas_call(
        _router_body,
        out_shape=[
            jax.ShapeDtypeStruct((T, 1), jnp.int32),
            jax.ShapeDtypeStruct((T, 1), jnp.int32),
            jax.ShapeDtypeStruct((T, 16), jnp.float32),
            jax.ShapeDtypeStruct((T, 16), jnp.float32),
            jax.ShapeDtypeStruct((NB + 1, 1), jnp.int32),
        ],
        interpret=interpret,
    )(x, wr)


def _gmm_body(gbt_ref, xs_ref, wg_ref, wu_ref, wd_ref, o_ref, wgc, wuc, last):
    j = pl.program_id(0)
    e = gbt_ref[1 + j]

    @pl.when((j < gbt_ref[0]) & ((j == 0) | (e != last[0])))
    def _():
        wgc[...] = wg_ref[0].astype(jnp.bfloat16)
        wuc[...] = wu_ref[0].astype(jnp.bfloat16)
        last[0] = e

    @pl.when(j < gbt_ref[0])
    def _():
        xb = xs_ref[...].astype(jnp.bfloat16)
        g = jnp.dot(xb, wgc[...], preferred_element_type=jnp.float32)
        u = jnp.dot(xb, wuc[...], preferred_element_type=jnp.float32)
        h = g * jax.nn.sigmoid(g) * u
        o_ref[...] = jnp.dot(h, wd_ref[0], preferred_element_type=jnp.float32)


def _gmm(gb, xs, wg, wu, wd, interpret=False):
    grid_spec = pltpu.PrefetchScalarGridSpec(
        num_scalar_prefetch=1,
        grid=(NB,),
        in_specs=[
            pl.BlockSpec((BM, D), lambda j, gbt: (j, 0)),
            pl.BlockSpec((1, D, F), lambda j, gbt: (gbt[1 + j], 0, 0)),
            pl.BlockSpec((1, D, F), lambda j, gbt: (gbt[1 + j], 0, 0)),
            pl.BlockSpec((1, F, D), lambda j, gbt: (gbt[1 + j], 0, 0)),
        ],
        out_specs=pl.BlockSpec((BM, D), lambda j, gbt: (j, 0)),
        scratch_shapes=[
            pltpu.VMEM((D, F), jnp.bfloat16),
            pltpu.VMEM((D, F), jnp.bfloat16),
            pltpu.SMEM((1,), jnp.int32),
        ],
    )
    return pl.pallas_call(
        _gmm_body,
        grid_spec=grid_spec,
        out_shape=jax.ShapeDtypeStruct((S, D), jnp.float32),
        compiler_params=pltpu.CompilerParams(
            dimension_semantics=("arbitrary",),
            vmem_limit_bytes=112 * 1024 * 1024),
        interpret=interpret,
    )(gb, xs, wg, wu, wd)


NW = 32
TPW = T // NW


def _nc():
    return plsc.get_sparse_core_info().num_cores


def _dispatch_scatter(x, pos0, pos1):
    nc = _nc()
    mesh = plsc.VectorSubcoreMesh(core_axis_name="c", subcore_axis_name="s")

    @functools.partial(
        pl.kernel,
        out_type=jax.ShapeDtypeStruct((S, D), jnp.float32),
        mesh=mesh,
        scratch_types=[
            pltpu.VMEM((TPW, D), jnp.float32),
            pltpu.VMEM((TPW,), jnp.int32),
            pltpu.VMEM((TPW,), jnp.int32),
            pltpu.SemaphoreType.DMA,
        ],
    )
    def k(x_hbm, pos0_hbm, pos1_hbm, xs_hbm, xbuf, i0, i1, sem):
        w = lax.axis_index("s") * nc + lax.axis_index("c")
        base = w * TPW
        pltpu.sync_copy(x_hbm.at[pl.ds(base, TPW)], xbuf)
        pltpu.sync_copy(pos0_hbm.at[pl.ds(base, TPW)], i0)
        pltpu.sync_copy(pos1_hbm.at[pl.ds(base, TPW)], i1)
        pltpu.async_copy(xbuf, xs_hbm.at[i0], sem).wait()
        pltpu.async_copy(xbuf, xs_hbm.at[i1], sem).wait()

    return k(x, pos0, pos1)


_CH = 32


def _combine(o, pos0, pos1, g0, g1):
    nc = _nc()
    mesh = plsc.VectorSubcoreMesh(core_axis_name="c", subcore_axis_name="s")

    @functools.partial(
        pl.kernel,
        out_type=jax.ShapeDtypeStruct((T, D), jnp.float32),
        mesh=mesh,
        scratch_types=[
            pltpu.VMEM((_CH, D), jnp.float32),
            pltpu.VMEM((_CH, D), jnp.float32),
            pltpu.VMEM((_CH,), jnp.int32),
            pltpu.VMEM((_CH,), jnp.int32),
            pltpu.VMEM((_CH, 16), jnp.float32),
            pltpu.VMEM((_CH, 16), jnp.float32),
            pltpu.SemaphoreType.DMA,
        ],
    )
    def k(o_hbm, pos0_hbm, pos1_hbm, g0_hbm, g1_hbm, out_hbm,
          b0, b1, i0, i1, gg0, gg1, sem):
        w = lax.axis_index("s") * nc + lax.axis_index("c")
        for cidx in range(TPW // _CH):
            base = w * TPW + cidx * _CH
            pltpu.sync_copy(pos0_hbm.at[pl.ds(base, _CH)], i0)
            pltpu.sync_copy(pos1_hbm.at[pl.ds(base, _CH)], i1)
            pltpu.sync_copy(g0_hbm.at[pl.ds(base, _CH)], gg0)
            pltpu.sync_copy(g1_hbm.at[pl.ds(base, _CH)], gg1)
            pltpu.async_copy(o_hbm.at[i0], b0, sem).wait()
            pltpu.async_copy(o_hbm.at[i1], b1, sem).wait()
            for tt in range(_CH):
                gv0 = gg0[tt, :]
                gv1 = gg1[tt, :]

                def body(jj, _, tt=tt, gv0=gv0, gv1=gv1):
                    sl = pl.ds(jj * 16, 16)
                    b0[tt, sl] = gv0 * b0[tt, sl] + gv1 * b1[tt, sl]
                    return 0

                lax.fori_loop(0, D // 16, body, 0)
            pltpu.sync_copy(b0, out_hbm.at[pl.ds(base, _CH)])

    return k(o, pos0, pos1, g0, g1)


def kernel(x_TD, w_router_DE, w_gate_EDF, w_up_EDF, w_down_EFD):
    pos0, pos1, g0, g1, gb = _router_dispatch(x_TD, w_router_DE)
    pos0 = pos0.reshape(T)
    pos1 = pos1.reshape(T)
    gb = gb.reshape(NB + 1)
    xs = _dispatch_scatter(x_TD, pos0, pos1)
    o = _gmm(gb, xs, w_gate_EDF, w_up_EDF, w_down_EFD)
    return _combine(o, pos0, pos1, g0, g1)


# speedup vs baseline: 1.0966x; 1.0250x over previous
import functools

import jax
import jax.numpy as jnp
from jax import lax
from jax.experimental import pallas as pl
from jax.experimental.pallas import tpu as pltpu
from jax.experimental.pallas import tpu_sc as plsc

T = 2048
D = 1024
F = 2048
E = 8
KTOP = 2

BM = 256
NB = T * KTOP // BM + E
S = NB * BM


def _router_body(x_ref, wr_ref, pos0_ref, pos1_ref, g0_ref, g1_ref, gb_ref):
    x = x_ref[...]
    logits = jnp.dot(x, wr_ref[...], preferred_element_type=jnp.float32)
    iota_e = lax.broadcasted_iota(jnp.int32, (T, E), 1)
    m0 = jnp.max(logits, axis=-1, keepdims=True)
    idx0 = jnp.min(jnp.where(logits == m0, iota_e, E), axis=-1, keepdims=True)
    masked = jnp.where(iota_e == idx0, -jnp.inf, logits)
    m1 = jnp.max(masked, axis=-1, keepdims=True)
    idx1 = jnp.min(jnp.where(masked == m1, iota_e, E), axis=-1, keepdims=True)
    d = jnp.exp(m1 - m0)
    g0_ref[...] = jnp.broadcast_to(1.0 / (1.0 + d), (T, 16))
    g1_ref[...] = jnp.broadcast_to(d / (1.0 + d), (T, 16))
    oh0 = (iota_e == idx0).astype(jnp.float32)
    oh1 = (iota_e == idx1).astype(jnp.float32)
    onehot = jnp.concatenate([oh0, oh1], axis=0)
    CH = 512
    r = lax.broadcasted_iota(jnp.int32, (CH, CH), 0)
    c = lax.broadcasted_iota(jnp.int32, (CH, CH), 1)
    tri = (r >= c).astype(jnp.float32)
    carry = jnp.zeros((1, E), dtype=jnp.float32)
    chunks = []
    for i in range(2 * T // CH):
        cs = jnp.dot(tri, onehot[i * CH:(i + 1) * CH], preferred_element_type=jnp.float32) + carry
        chunks.append(cs)
        carry = cs[CH - 1:CH, :]
    ccum = jnp.concatenate(chunks, axis=0)
    counts = ccum[2 * T - 1:2 * T, :]
    nb = jnp.floor((counts + (BM - 1)) * (1.0 / BM))
    r8 = lax.broadcasted_iota(jnp.int32, (E, E), 0)
    c8 = lax.broadcasted_iota(jnp.int32, (E, E), 1)
    m_lt = (r8 < c8).astype(jnp.float32)
    m_le = (r8 <= c8).astype(jnp.float32)
    bases = BM * jnp.dot(nb, m_lt, preferred_element_type=jnp.float32)
    incl_nb = jnp.dot(nb, m_le, preferred_element_type=jnp.float32)
    pos = jnp.sum(onehot * (ccum - 1.0 + bases), axis=-1, keepdims=True)
    pos = pos.astype(jnp.int32)
    pos0_ref[...] = pos[:T]
    pos1_ref[...] = pos[T:]
    jmat = lax.broadcasted_iota(jnp.int32, (NB, E), 0).astype(jnp.float32)
    gb = jnp.sum((jnp.broadcast_to(incl_nb, (NB, E)) <= jmat).astype(jnp.float32),
                 axis=-1, keepdims=True)
    gb_ref[0:1] = incl_nb[:, E - 1:E].astype(jnp.int32)
    gb_ref[1:] = jnp.minimum(gb, E - 1).astype(jnp.int32)


def _router_dispatch(x, wr, interpret=False):
    return pl.pallas_call(
        _router_body,
        out_shape=[
            jax.ShapeDtypeStruct((T, 1), jnp.int32),
            jax.ShapeDtypeStruct((T, 1), jnp.int32),
            jax.ShapeDtypeStruct((T, 16), jnp.float32),
            jax.ShapeDtypeStruct((T, 16), jnp.float32),
            jax.ShapeDtypeStruct((NB + 1, 1), jnp.int32),
        ],
        interpret=interpret,
    )(x, wr)


def _gmm_body(gbt_ref, xs_ref, wg_ref, wu_ref, wd_ref, o_ref, wgc, wuc, last):
    j = pl.program_id(0)
    e = gbt_ref[1 + j]

    @pl.when((j < gbt_ref[0]) & ((j == 0) | (e != last[0])))
    def _():
        wgc[...] = wg_ref[0].astype(jnp.bfloat16)
        wuc[...] = wu_ref[0].astype(jnp.bfloat16)
        last[0] = e

    @pl.when(j < gbt_ref[0])
    def _():
        xb = xs_ref[...].astype(jnp.bfloat16)
        g = jnp.dot(xb, wgc[...], preferred_element_type=jnp.float32)
        u = jnp.dot(xb, wuc[...], preferred_element_type=jnp.float32)
        h = g * jax.nn.sigmoid(g) * u
        o_ref[...] = jnp.dot(h, wd_ref[0], preferred_element_type=jnp.float32)


def _gmm(gb, xs, wg, wu, wd, interpret=False):
    grid_spec = pltpu.PrefetchScalarGridSpec(
        num_scalar_prefetch=1,
        grid=(NB,),
        in_specs=[
            pl.BlockSpec((BM, D), lambda j, gbt: (j, 0)),
            pl.BlockSpec((1, D, F), lambda j, gbt: (gbt[1 + j], 0, 0)),
            pl.BlockSpec((1, D, F), lambda j, gbt: (gbt[1 + j], 0, 0)),
            pl.BlockSpec((1, F, D), lambda j, gbt: (gbt[1 + j], 0, 0)),
        ],
        out_specs=pl.BlockSpec((BM, D), lambda j, gbt: (j, 0)),
        scratch_shapes=[
            pltpu.VMEM((D, F), jnp.bfloat16),
            pltpu.VMEM((D, F), jnp.bfloat16),
            pltpu.SMEM((1,), jnp.int32),
        ],
    )
    return pl.pallas_call(
        _gmm_body,
        grid_spec=grid_spec,
        out_shape=jax.ShapeDtypeStruct((S, D), jnp.float32),
        compiler_params=pltpu.CompilerParams(
            dimension_semantics=("arbitrary",),
            vmem_limit_bytes=112 * 1024 * 1024),
        interpret=interpret,
    )(gb, xs, wg, wu, wd)


NW = 32
TPW = T // NW


def _nc():
    return plsc.get_sparse_core_info().num_cores


def _dispatch_scatter(x, pos0, pos1):
    nc = _nc()
    mesh = plsc.VectorSubcoreMesh(core_axis_name="c", subcore_axis_name="s")

    @functools.partial(
        pl.kernel,
        out_type=jax.ShapeDtypeStruct((S, D), jnp.float32),
        mesh=mesh,
        scratch_types=[
            pltpu.VMEM((TPW, D), jnp.float32),
            pltpu.VMEM((TPW,), jnp.int32),
            pltpu.VMEM((TPW,), jnp.int32),
            pltpu.SemaphoreType.DMA,
        ],
    )
    def k(x_hbm, pos0_hbm, pos1_hbm, xs_hbm, xbuf, i0, i1, sem):
        w = lax.axis_index("s") * nc + lax.axis_index("c")
        base = w * TPW
        pltpu.sync_copy(x_hbm.at[pl.ds(base, TPW)], xbuf)
        pltpu.sync_copy(pos0_hbm.at[pl.ds(base, TPW)], i0)
        pltpu.sync_copy(pos1_hbm.at[pl.ds(base, TPW)], i1)
        c0 = pltpu.async_copy(xbuf, xs_hbm.at[i0], sem)
        c1 = pltpu.async_copy(xbuf, xs_hbm.at[i1], sem)
        c0.wait()
        c1.wait()

    return k(x, pos0, pos1)


_CH = 32


def _combine(o, pos0, pos1, g0, g1):
    nc = _nc()
    mesh = plsc.VectorSubcoreMesh(core_axis_name="c", subcore_axis_name="s")

    @functools.partial(
        pl.kernel,
        out_type=jax.ShapeDtypeStruct((T, D), jnp.float32),
        mesh=mesh,
        scratch_types=[
            pltpu.VMEM((_CH, D), jnp.float32),
            pltpu.VMEM((_CH, D), jnp.float32),
            pltpu.VMEM((_CH,), jnp.int32),
            pltpu.VMEM((_CH,), jnp.int32),
            pltpu.VMEM((_CH, 16), jnp.float32),
            pltpu.VMEM((_CH, 16), jnp.float32),
            pltpu.SemaphoreType.DMA,
        ],
    )
    def k(o_hbm, pos0_hbm, pos1_hbm, g0_hbm, g1_hbm, out_hbm,
          b0, b1, i0, i1, gg0, gg1, sem):
        w = lax.axis_index("s") * nc + lax.axis_index("c")
        for cidx in range(TPW // _CH):
            base = w * TPW + cidx * _CH
            pltpu.sync_copy(pos0_hbm.at[pl.ds(base, _CH)], i0)
            pltpu.sync_copy(pos1_hbm.at[pl.ds(base, _CH)], i1)
            pltpu.sync_copy(g0_hbm.at[pl.ds(base, _CH)], gg0)
            pltpu.sync_copy(g1_hbm.at[pl.ds(base, _CH)], gg1)
            c0 = pltpu.async_copy(o_hbm.at[i0], b0, sem)
            c1 = pltpu.async_copy(o_hbm.at[i1], b1, sem)
            c0.wait()
            c1.wait()
            for tt in range(_CH):
                gv0 = gg0[tt, :]
                gv1 = gg1[tt, :]

                def body(jj, _, tt=tt, gv0=gv0, gv1=gv1):
                    col = jj * 64
                    for u in range(4):
                        sl = pl.ds(col + u * 16, 16)
                        b0[tt, sl] = gv0 * b0[tt, sl] + gv1 * b1[tt, sl]
                    return 0

                lax.fori_loop(0, D // 64, body, 0)
            pltpu.sync_copy(b0, out_hbm.at[pl.ds(base, _CH)])

    return k(o, pos0, pos1, g0, g1)


def kernel(x_TD, w_router_DE, w_gate_EDF, w_up_EDF, w_down_EFD):
    pos0, pos1, g0, g1, gb = _router_dispatch(x_TD, w_router_DE)
    pos0 = pos0.reshape(T)
    pos1 = pos1.reshape(T)
    gb = gb.reshape(NB + 1)
    xs = _dispatch_scatter(x_TD, pos0, pos1)
    o = _gmm(gb, xs, w_gate_EDF, w_up_EDF, w_down_EFD)
    return _combine(o, pos0, pos1, g0, g1)


# speedup vs baseline: 1.2794x; 1.1666x over previous
import functools

import jax
import jax.numpy as jnp
from jax import lax
from jax.experimental import pallas as pl
from jax.experimental.pallas import tpu as pltpu
from jax.experimental.pallas import tpu_sc as plsc

T = 2048
D = 1024
F = 2048
E = 8
KTOP = 2

BM = 256
NB = T * KTOP // BM + E
S = NB * BM


def _router_body(x_ref, wr_ref, pos0_ref, pos1_ref, g0_ref, g1_ref, gb_ref):
    x = x_ref[...]
    logits = jnp.dot(x, wr_ref[...], preferred_element_type=jnp.float32)
    iota_e = lax.broadcasted_iota(jnp.int32, (T, E), 1)
    m0 = jnp.max(logits, axis=-1, keepdims=True)
    idx0 = jnp.min(jnp.where(logits == m0, iota_e, E), axis=-1, keepdims=True)
    masked = jnp.where(iota_e == idx0, -jnp.inf, logits)
    m1 = jnp.max(masked, axis=-1, keepdims=True)
    idx1 = jnp.min(jnp.where(masked == m1, iota_e, E), axis=-1, keepdims=True)
    d = jnp.exp(m1 - m0)
    g0_ref[...] = jnp.broadcast_to(1.0 / (1.0 + d), (T, 16))
    g1_ref[...] = jnp.broadcast_to(d / (1.0 + d), (T, 16))
    oh0 = (iota_e == idx0).astype(jnp.float32)
    oh1 = (iota_e == idx1).astype(jnp.float32)
    onehot = jnp.concatenate([oh0, oh1], axis=0)
    CH = 512
    r = lax.broadcasted_iota(jnp.int32, (CH, CH), 0)
    c = lax.broadcasted_iota(jnp.int32, (CH, CH), 1)
    tri = (r >= c).astype(jnp.float32)
    carry = jnp.zeros((1, E), dtype=jnp.float32)
    chunks = []
    for i in range(2 * T // CH):
        cs = jnp.dot(tri, onehot[i * CH:(i + 1) * CH], preferred_element_type=jnp.float32) + carry
        chunks.append(cs)
        carry = cs[CH - 1:CH, :]
    ccum = jnp.concatenate(chunks, axis=0)
    counts = ccum[2 * T - 1:2 * T, :]
    nb = jnp.floor((counts + (BM - 1)) * (1.0 / BM))
    r8 = lax.broadcasted_iota(jnp.int32, (E, E), 0)
    c8 = lax.broadcasted_iota(jnp.int32, (E, E), 1)
    m_lt = (r8 < c8).astype(jnp.float32)
    m_le = (r8 <= c8).astype(jnp.float32)
    bases = BM * jnp.dot(nb, m_lt, preferred_element_type=jnp.float32)
    incl_nb = jnp.dot(nb, m_le, preferred_element_type=jnp.float32)
    pos = jnp.sum(onehot * (ccum - 1.0 + bases), axis=-1, keepdims=True)
    pos = pos.astype(jnp.int32)
    pos0_ref[...] = pos[:T]
    pos1_ref[...] = pos[T:]
    jmat = lax.broadcasted_iota(jnp.int32, (NB, E), 0).astype(jnp.float32)
    gb = jnp.sum((jnp.broadcast_to(incl_nb, (NB, E)) <= jmat).astype(jnp.float32),
                 axis=-1, keepdims=True)
    gb_ref[0:1] = incl_nb[:, E - 1:E].astype(jnp.int32)
    gb_ref[1:] = jnp.minimum(gb, E - 1).astype(jnp.int32)


def _router_dispatch(x, wr, interpret=False):
    return pl.pallas_call(
        _router_body,
        out_shape=[
            jax.ShapeDtypeStruct((T, 1), jnp.int32),
            jax.ShapeDtypeStruct((T, 1), jnp.int32),
            jax.ShapeDtypeStruct((T, 16), jnp.float32),
            jax.ShapeDtypeStruct((T, 16), jnp.float32),
            jax.ShapeDtypeStruct((NB + 1, 1), jnp.int32),
        ],
        interpret=interpret,
    )(x, wr)


def _gmm_body(meta_ref, xs_ref, wg_hbm, wu_hbm, wd_hbm, o_ref,
              wb, ub, db, sem0, sem1):
    j = pl.program_id(0)
    tot = meta_ref[0]
    e = meta_ref[1 + j]
    first = meta_ref[1 + NB + j]
    slot = meta_ref[1 + 2 * NB + j]
    nxte = meta_ref[1 + 3 * NB + j]

    def _copies(eidx, s, sem):
        return (pltpu.make_async_copy(wg_hbm.at[eidx], wb.at[s], sem),
                pltpu.make_async_copy(wu_hbm.at[eidx], ub.at[s], sem),
                pltpu.make_async_copy(wd_hbm.at[eidx], db.at[s], sem))

    @pl.when(j == 0)
    def _():
        for c in _copies(e, 0, sem0):
            c.start()

    @pl.when((first == 1) & (nxte >= 0) & (slot == 0))
    def _():
        for c in _copies(nxte, 1, sem1):
            c.start()

    @pl.when((first == 1) & (nxte >= 0) & (slot == 1))
    def _():
        for c in _copies(nxte, 0, sem0):
            c.start()

    @pl.when((first == 1) & (slot == 0))
    def _():
        for c in _copies(e, 0, sem0):
            c.wait()

    @pl.when((first == 1) & (slot == 1))
    def _():
        for c in _copies(e, 1, sem1):
            c.wait()

    @pl.when(j < tot)
    def _():
        xb = xs_ref[...]
        g = jnp.dot(xb, wb[slot], preferred_element_type=jnp.float32)
        u = jnp.dot(xb, ub[slot], preferred_element_type=jnp.float32)
        h = g * jax.nn.sigmoid(g) * u
        o_ref[...] = jnp.dot(h, db[slot], preferred_element_type=jnp.float32)


def _gmm(meta, xs, wg, wu, wd, interpret=False):
    grid_spec = pltpu.PrefetchScalarGridSpec(
        num_scalar_prefetch=1,
        grid=(NB,),
        in_specs=[
            pl.BlockSpec((BM, D), lambda j, meta: (j, 0)),
            pl.BlockSpec(memory_space=pl.ANY),
            pl.BlockSpec(memory_space=pl.ANY),
            pl.BlockSpec(memory_space=pl.ANY),
        ],
        out_specs=pl.BlockSpec((BM, D), lambda j, meta: (j, 0)),
        scratch_shapes=[
            pltpu.VMEM((2, D, F), jnp.float32),
            pltpu.VMEM((2, D, F), jnp.float32),
            pltpu.VMEM((2, F, D), jnp.float32),
            pltpu.SemaphoreType.DMA,
            pltpu.SemaphoreType.DMA,
        ],
    )
    return pl.pallas_call(
        _gmm_body,
        grid_spec=grid_spec,
        out_shape=jax.ShapeDtypeStruct((S, D), jnp.float32),
        compiler_params=pltpu.CompilerParams(
            dimension_semantics=("arbitrary",)),
        interpret=interpret,
    )(meta, xs, wg, wu, wd)


def _gmm_meta(gb):
    tot = gb[0:1]
    gbv = gb[1:]
    prev = jnp.concatenate([jnp.full((1,), -1, jnp.int32), gbv[:-1]])
    first = (gbv != prev).astype(jnp.int32)
    run_id = jnp.cumsum(first) - 1
    slot = run_id % 2
    nruns = run_id[-1] + 1
    sc_idx = jnp.where(first == 1, run_id, NB)
    rexp = jnp.zeros((NB,), jnp.int32).at[sc_idx].set(gbv, mode='drop')
    nxt_shift = jnp.concatenate([rexp[1:], jnp.full((1,), -1, jnp.int32)])
    nxte_run = jnp.where(jnp.arange(NB) + 1 < nruns, nxt_shift, -1)
    nxte = nxte_run[run_id]
    return jnp.concatenate([tot, gbv, first, slot, nxte])


NW = 32
TPW = T // NW


def _nc():
    return plsc.get_sparse_core_info().num_cores


def _dispatch_scatter(x, pos0, pos1):
    nc = _nc()
    mesh = plsc.VectorSubcoreMesh(core_axis_name="c", subcore_axis_name="s")

    @functools.partial(
        pl.kernel,
        out_type=jax.ShapeDtypeStruct((S, D), jnp.float32),
        mesh=mesh,
        scratch_types=[
            pltpu.VMEM((TPW, D), jnp.float32),
            pltpu.VMEM((TPW,), jnp.int32),
            pltpu.VMEM((TPW,), jnp.int32),
            pltpu.SemaphoreType.DMA,
        ],
    )
    def k(x_hbm, pos0_hbm, pos1_hbm, xs_hbm, xbuf, i0, i1, sem):
        w = lax.axis_index("s") * nc + lax.axis_index("c")
        base = w * TPW
        pltpu.sync_copy(x_hbm.at[pl.ds(base, TPW)], xbuf)
        pltpu.sync_copy(pos0_hbm.at[pl.ds(base, TPW)], i0)
        pltpu.sync_copy(pos1_hbm.at[pl.ds(base, TPW)], i1)
        c0 = pltpu.async_copy(xbuf, xs_hbm.at[i0], sem)
        c1 = pltpu.async_copy(xbuf, xs_hbm.at[i1], sem)
        c0.wait()
        c1.wait()

    return k(x, pos0, pos1)


_CH = 32


def _combine(o, pos0, pos1, g0, g1):
    nc = _nc()
    mesh = plsc.VectorSubcoreMesh(core_axis_name="c", subcore_axis_name="s")

    @functools.partial(
        pl.kernel,
        out_type=jax.ShapeDtypeStruct((T, D), jnp.float32),
        mesh=mesh,
        scratch_types=[
            pltpu.VMEM((_CH, D), jnp.float32),
            pltpu.VMEM((_CH, D), jnp.float32),
            pltpu.VMEM((_CH,), jnp.int32),
            pltpu.VMEM((_CH,), jnp.int32),
            pltpu.VMEM((_CH, 16), jnp.float32),
            pltpu.VMEM((_CH, 16), jnp.float32),
            pltpu.SemaphoreType.DMA,
        ],
    )
    def k(o_hbm, pos0_hbm, pos1_hbm, g0_hbm, g1_hbm, out_hbm,
          b0, b1, i0, i1, gg0, gg1, sem):
        w = lax.axis_index("s") * nc + lax.axis_index("c")
        for cidx in range(TPW // _CH):
            base = w * TPW + cidx * _CH
            pltpu.sync_copy(pos0_hbm.at[pl.ds(base, _CH)], i0)
            pltpu.sync_copy(pos1_hbm.at[pl.ds(base, _CH)], i1)
            pltpu.sync_copy(g0_hbm.at[pl.ds(base, _CH)], gg0)
            pltpu.sync_copy(g1_hbm.at[pl.ds(base, _CH)], gg1)
            c0 = pltpu.async_copy(o_hbm.at[i0], b0, sem)
            c1 = pltpu.async_copy(o_hbm.at[i1], b1, sem)
            c0.wait()
            c1.wait()
            for tt in range(_CH):
                gv0 = gg0[tt, :]
                gv1 = gg1[tt, :]

                def body(jj, _, tt=tt, gv0=gv0, gv1=gv1):
                    col = jj * 64
                    for u in range(4):
                        sl = pl.ds(col + u * 16, 16)
                        b0[tt, sl] = gv0 * b0[tt, sl] + gv1 * b1[tt, sl]
                    return 0

                lax.fori_loop(0, D // 64, body, 0)
            pltpu.sync_copy(b0, out_hbm.at[pl.ds(base, _CH)])

    return k(o, pos0, pos1, g0, g1)


def kernel(x_TD, w_router_DE, w_gate_EDF, w_up_EDF, w_down_EFD):
    pos0, pos1, g0, g1, gb = _router_dispatch(x_TD, w_router_DE)
    pos0 = pos0.reshape(T)
    pos1 = pos1.reshape(T)
    meta = _gmm_meta(gb.reshape(NB + 1))
    xs = _dispatch_scatter(x_TD, pos0, pos1)
    o = _gmm(meta, xs, w_gate_EDF, w_up_EDF, w_down_EFD)
    return _combine(o, pos0, pos1, g0, g1)
